# baseline (speedup 1.0000x reference)
import functools

import jax
import jax.numpy as jnp
from jax import lax
from jax.experimental import pallas as pl
from jax.experimental.pallas import tpu as pltpu
from jax.experimental.pallas import tpu_sc as plsc

NC = 2
NS = 16
NW = NC * NS

EMB = 64
VOCAB = 1000000
VBLK = 128
NFULL = VOCAB // VBLK
IN_STRIDE = 137

BGRP = 4
NBUF = 4

_mesh = plsc.VectorSubcoreMesh(core_axis_name="c", subcore_axis_name="s")
_params = pltpu.CompilerParams(use_tc_tiling_on_sc=True)
_params_nl = pltpu.CompilerParams(use_tc_tiling_on_sc=True, needs_layout_passes=False)


@functools.partial(
    pl.kernel,
    mesh=_mesh,
    compiler_params=_params_nl,
    out_type=jax.ShapeDtypeStruct((VOCAB, 128), jnp.float32),
    scratch_types=[
        [pltpu.VMEM((EMB, IN_STRIDE), jnp.float32) for _ in range(2)],
        [pltpu.VMEM((VBLK, 128), jnp.float32) for _ in range(2)],
        [pltpu.SemaphoreType.DMA for _ in range(2)],
        [pltpu.SemaphoreType.DMA for _ in range(2)],
    ],
)
def _build_fat(tabt_hbm, tail_hbm, fat_hbm, inb, outb, isem, osem):
    wid = lax.axis_index("s") * NC + lax.axis_index("c")
    nblk = (NFULL - wid + NW - 1) // NW

    def load(j, p):
        v0 = (wid + j * NW) * VBLK
        pltpu.async_copy(
            tabt_hbm.at[:, pl.ds(v0, VBLK)], inb[p].at[:, pl.ds(0, VBLK)],
            isem[p],
        )

    def lwait(p):
        pltpu.make_async_copy(
            tabt_hbm.at[:, pl.ds(0, VBLK)], inb[p].at[:, pl.ds(0, VBLK)],
            isem[p],
        ).wait()

    def store(j, p):
        v0 = (wid + j * NW) * VBLK
        pltpu.async_copy(outb[p], fat_hbm.at[pl.ds(v0, VBLK)], osem[p])

    def swait(p):
        pltpu.make_async_copy(
            outb[p], fat_hbm.at[pl.ds(0, VBLK)], osem[p]
        ).wait()

    def transpose(p):
        lanes = lax.iota(jnp.int32, 16)
        for m in range(EMB // 16):
            eidx = lanes + m * 16

            def vrow(v, carry):
                vcol = jnp.full((16,), v, jnp.int32)
                vals = plsc.load_gather(inb[p], [eidx, vcol])
                outb[p][v, pl.ds(m * 16, 16)] = vals
                return carry

            lax.fori_loop(0, VBLK, vrow, 0, unroll=4)

    @pl.when(wid == 0)
    def _():
        pltpu.sync_copy(tail_hbm, fat_hbm.at[pl.ds(NFULL * VBLK, 64)])

    load(0, 0)

    def half(j, par):
        @pl.when(j < nblk)
        def _():
            @pl.when(j + 1 < nblk)
            def _():
                load(j + 1, 1 - par)

            lwait(par)

            @pl.when(j >= 2)
            def _():
                swait(par)

            transpose(par)
            store(j, par)

    def body(jj, carry):
        half(2 * jj, 0)
        half(2 * jj + 1, 1)
        return carry

    lax.fori_loop(0, (nblk + 1) // 2, body, 0)
    swait(0)
    swait(1)


def _make_gather(B: int, L: int):
    b_per_w = B // NW
    chunk = BGRP * L
    n_chunks = b_per_w // BGRP
    n_super = n_chunks // NBUF

    @functools.partial(
        pl.kernel,
        mesh=_mesh,
        compiler_params=_params,
        out_type=jax.ShapeDtypeStruct((B, L, 128), jnp.float32),
        scratch_types=[
            pltpu.VMEM((b_per_w * L,), jnp.int32),
            [pltpu.VMEM((chunk, 128), jnp.float32) for _ in range(NBUF)],
            [pltpu.SemaphoreType.DMA for _ in range(NBUF)],
            [pltpu.SemaphoreType.DMA for _ in range(NBUF)],
        ],
    )
    def lookup(x_hbm, table_hbm, out_hbm, idx_v, rows, gsem, wsem):
        wid = lax.axis_index("s") * NC + lax.axis_index("c")
        base_b = wid * b_per_w
        pltpu.sync_copy(x_hbm.at[wid], idx_v)

        def gather(j, b):
            pltpu.async_copy(
                table_hbm.at[idx_v.at[pl.ds(j * chunk, chunk)]], rows[b], gsem[b]
            )

        def write(j, b):
            for g in range(BGRP):
                pltpu.async_copy(
                    rows[b].at[pl.ds(g * L, L), :],
                    out_hbm.at[base_b + j * BGRP + g],
                    wsem[b],
                )

        def gwait(b):
            pltpu.make_async_copy(
                table_hbm.at[pl.ds(0, chunk)], rows[b], gsem[b]
            ).wait()

        def wwait(b):
            for g in range(BGRP):
                pltpu.make_async_copy(
                    rows[b].at[pl.ds(g * L, L), :],
                    out_hbm.at[base_b + g],
                    wsem[b],
                ).wait()

        for b in range(NBUF):
            gather(b, b)

        def body(si, carry):
            for b in range(NBUF):
                j = si * NBUF + b
                gwait(b)
                write(j, b)
                wwait(b)
                gather(j + NBUF, b)
            return carry

        lax.fori_loop(0, n_super - 1, body, 0)

        for b in range(NBUF):
            j = (n_super - 1) * NBUF + b
            gwait(b)
            write(j, b)
        for b in range(NBUF):
            wwait(b)

    return lookup


def kernel(x, table):
    B, L = x.shape
    x_flat = x.reshape(NW, B // NW * L)
    tail_fat = jnp.pad(table[NFULL * VBLK :], ((0, 0), (0, 128 - EMB)))
    table_fat = _build_fat(table.T, tail_fat)
    out_fat = _make_gather(B, L)(x_flat, table_fat)
    return out_fat[:, :, :EMB]


# speedup vs baseline: 2.0574x; 2.0574x over previous
import functools

import jax
import jax.numpy as jnp
from jax import lax
from jax.experimental import pallas as pl
from jax.experimental.pallas import tpu as pltpu
from jax.experimental.pallas import tpu_sc as plsc

NC = 2
NS = 16
NW = NC * NS

EMB = 64
VOCAB = 1000000
VB = 2048

BGRP = 4
NBUF = 4

_mesh = plsc.VectorSubcoreMesh(core_axis_name="c", subcore_axis_name="s")
_params = pltpu.CompilerParams(use_tc_tiling_on_sc=True)


def _fat_block(tabt_ref, fat_ref):
    fat_ref[:, 0:EMB] = tabt_ref[...].T


def _build_fat(tabt):
    grid = pl.cdiv(VOCAB, VB)
    return pl.pallas_call(
        _fat_block,
        grid=(grid,),
        in_specs=[pl.BlockSpec((EMB, VB), lambda i: (0, i))],
        out_specs=pl.BlockSpec((VB, 128), lambda i: (i, 0)),
        out_shape=jax.ShapeDtypeStruct((VOCAB, 128), jnp.float32),
    )(tabt)


def _make_gather(B: int, L: int):
    b_per_w = B // NW
    chunk = BGRP * L
    n_chunks = b_per_w // BGRP
    n_super = n_chunks // NBUF

    @functools.partial(
        pl.kernel,
        mesh=_mesh,
        compiler_params=_params,
        out_type=jax.ShapeDtypeStruct((B, L, 128), jnp.float32),
        scratch_types=[
            pltpu.VMEM((b_per_w * L,), jnp.int32),
            [pltpu.VMEM((chunk, 128), jnp.float32) for _ in range(NBUF)],
            [pltpu.SemaphoreType.DMA for _ in range(NBUF)],
            [pltpu.SemaphoreType.DMA for _ in range(NBUF)],
        ],
    )
    def lookup(x_hbm, table_hbm, out_hbm, idx_v, rows, gsem, wsem):
        wid = lax.axis_index("s") * NC + lax.axis_index("c")
        base_b = wid * b_per_w
        pltpu.sync_copy(x_hbm.at[wid], idx_v)

        def gather(j, b):
            pltpu.async_copy(
                table_hbm.at[idx_v.at[pl.ds(j * chunk, chunk)]], rows[b], gsem[b]
            )

        def write(j, b):
            for g in range(BGRP):
                pltpu.async_copy(
                    rows[b].at[pl.ds(g * L, L), :],
                    out_hbm.at[base_b + j * BGRP + g],
                    wsem[b],
                )

        def gwait(b):
            pltpu.make_async_copy(
                table_hbm.at[pl.ds(0, chunk)], rows[b], gsem[b]
            ).wait()

        def wwait(b):
            for g in range(BGRP):
                pltpu.make_async_copy(
                    rows[b].at[pl.ds(g * L, L), :],
                    out_hbm.at[base_b + g],
                    wsem[b],
                ).wait()

        for b in range(NBUF):
            gather(b, b)

        def body(si, carry):
            for b in range(NBUF):
                j = si * NBUF + b
                gwait(b)
                write(j, b)
                wwait(b)
                gather(j + NBUF, b)
            return carry

        lax.fori_loop(0, n_super - 1, body, 0)

        for b in range(NBUF):
            j = (n_super - 1) * NBUF + b
            gwait(b)
            write(j, b)
        for b in range(NBUF):
            wwait(b)

    return lookup


def kernel(x, table):
    B, L = x.shape
    x_flat = x.reshape(NW, B // NW * L)
    table_fat = _build_fat(table.T)
    out_fat = _make_gather(B, L)(x_flat, table_fat)
    return out_fat[:, :, :EMB]


# speedup vs baseline: 2.5796x; 1.2538x over previous
import functools

import jax
import jax.numpy as jnp
from jax import lax
from jax.experimental import pallas as pl
from jax.experimental.pallas import tpu as pltpu
from jax.experimental.pallas import tpu_sc as plsc

NC = 2
NS = 16
NW = NC * NS

EMB = 64
VOCAB = 1000000
VB = 8192

BGRP = 4
NBUF = 4

_mesh = plsc.VectorSubcoreMesh(core_axis_name="c", subcore_axis_name="s")
_params = pltpu.CompilerParams(use_tc_tiling_on_sc=True)


def _fat_block(tabt_ref, fat_ref):
    fat_ref[:, 0:EMB] = tabt_ref[...].T


def _build_fat(tabt):
    grid = pl.cdiv(VOCAB, VB)
    return pl.pallas_call(
        _fat_block,
        grid=(grid,),
        in_specs=[pl.BlockSpec((EMB, VB), lambda i: (0, i))],
        out_specs=pl.BlockSpec((VB, 128), lambda i: (i, 0)),
        out_shape=jax.ShapeDtypeStruct((VOCAB, 128), jnp.float32),
    )(tabt)


def _make_gather(B: int, L: int):
    b_per_w = B // NW
    chunk = BGRP * L
    n_chunks = b_per_w // BGRP
    n_super = n_chunks // NBUF

    @functools.partial(
        pl.kernel,
        mesh=_mesh,
        compiler_params=_params,
        out_type=jax.ShapeDtypeStruct((B, L, 128), jnp.float32),
        scratch_types=[
            pltpu.VMEM((b_per_w * L,), jnp.int32),
            [pltpu.VMEM((chunk, 128), jnp.float32) for _ in range(NBUF)],
            [pltpu.SemaphoreType.DMA for _ in range(NBUF)],
            [pltpu.SemaphoreType.DMA for _ in range(NBUF)],
        ],
    )
    def lookup(x_hbm, table_hbm, out_hbm, idx_v, rows, gsem, wsem):
        wid = lax.axis_index("s") * NC + lax.axis_index("c")
        base_b = wid * b_per_w
        pltpu.sync_copy(x_hbm.at[wid], idx_v)

        def gather(j, b):
            pltpu.async_copy(
                table_hbm.at[idx_v.at[pl.ds(j * chunk, chunk)]], rows[b], gsem[b]
            )

        def write(j, b):
            for g in range(BGRP):
                pltpu.async_copy(
                    rows[b].at[pl.ds(g * L, L), :],
                    out_hbm.at[base_b + j * BGRP + g],
                    wsem[b],
                )

        def gwait(b):
            pltpu.make_async_copy(
                table_hbm.at[pl.ds(0, chunk)], rows[b], gsem[b]
            ).wait()

        def wwait(b):
            for g in range(BGRP):
                pltpu.make_async_copy(
                    rows[b].at[pl.ds(g * L, L), :],
                    out_hbm.at[base_b + g],
                    wsem[b],
                ).wait()

        for b in range(NBUF):
            gather(b, b)

        def body(si, carry):
            for b in range(NBUF):
                j = si * NBUF + b
                gwait(b)
                write(j, b)
                wwait(b)
                gather(j + NBUF, b)
            return carry

        lax.fori_loop(0, n_super - 1, body, 0)

        for b in range(NBUF):
            j = (n_super - 1) * NBUF + b
            gwait(b)
            write(j, b)
        for b in range(NBUF):
            wwait(b)

    return lookup


def kernel(x, table):
    B, L = x.shape
    x_flat = x.reshape(NW, B // NW * L)
    table_fat = _build_fat(table.T)
    out_fat = _make_gather(B, L)(x_flat, table_fat)
    return out_fat[:, :, :EMB]


# speedup vs baseline: 2.6548x; 1.0291x over previous
import functools

import jax
import jax.numpy as jnp
from jax import lax
from jax.experimental import pallas as pl
from jax.experimental.pallas import tpu as pltpu
from jax.experimental.pallas import tpu_sc as plsc

NC = 2
NS = 16
NW = NC * NS

EMB = 64
VOCAB = 1000000
VB = 16384

BGRP = 4
NBUF = 4

_mesh = plsc.VectorSubcoreMesh(core_axis_name="c", subcore_axis_name="s")
_params = pltpu.CompilerParams(use_tc_tiling_on_sc=True)


def _fat_block(tabt_ref, fat_ref):
    fat_ref[:, 0:EMB] = tabt_ref[...].T


def _build_fat(tabt):
    grid = pl.cdiv(VOCAB, VB)
    return pl.pallas_call(
        _fat_block,
        grid=(grid,),
        in_specs=[pl.BlockSpec((EMB, VB), lambda i: (0, i))],
        out_specs=pl.BlockSpec((VB, 128), lambda i: (i, 0)),
        out_shape=jax.ShapeDtypeStruct((VOCAB, 128), jnp.float32),
    )(tabt)


def _make_gather(B: int, L: int):
    b_per_w = B // NW
    chunk = BGRP * L
    n_chunks = b_per_w // BGRP
    n_super = n_chunks // NBUF

    @functools.partial(
        pl.kernel,
        mesh=_mesh,
        compiler_params=_params,
        out_type=jax.ShapeDtypeStruct((B, L, 128), jnp.float32),
        scratch_types=[
            pltpu.VMEM((b_per_w * L,), jnp.int32),
            [pltpu.VMEM((chunk, 128), jnp.float32) for _ in range(NBUF)],
            [pltpu.SemaphoreType.DMA for _ in range(NBUF)],
            [pltpu.SemaphoreType.DMA for _ in range(NBUF)],
        ],
    )
    def lookup(x_hbm, table_hbm, out_hbm, idx_v, rows, gsem, wsem):
        wid = lax.axis_index("s") * NC + lax.axis_index("c")
        base_b = wid * b_per_w
        pltpu.sync_copy(x_hbm.at[wid], idx_v)

        def gather(j, b):
            pltpu.async_copy(
                table_hbm.at[idx_v.at[pl.ds(j * chunk, chunk)]], rows[b], gsem[b]
            )

        def write(j, b):
            for g in range(BGRP):
                pltpu.async_copy(
                    rows[b].at[pl.ds(g * L, L), :],
                    out_hbm.at[base_b + j * BGRP + g],
                    wsem[b],
                )

        def gwait(b):
            pltpu.make_async_copy(
                table_hbm.at[pl.ds(0, chunk)], rows[b], gsem[b]
            ).wait()

        def wwait(b):
            for g in range(BGRP):
                pltpu.make_async_copy(
                    rows[b].at[pl.ds(g * L, L), :],
                    out_hbm.at[base_b + g],
                    wsem[b],
                ).wait()

        for b in range(NBUF):
            gather(b, b)

        def body(si, carry):
            for b in range(NBUF):
                j = si * NBUF + b
                gwait(b)
                write(j, b)
                wwait(b)
                gather(j + NBUF, b)
            return carry

        lax.fori_loop(0, n_super - 1, body, 0)

        for b in range(NBUF):
            j = (n_super - 1) * NBUF + b
            gwait(b)
            write(j, b)
        for b in range(NBUF):
            wwait(b)

    return lookup


def kernel(x, table):
    B, L = x.shape
    x_flat = x.reshape(NW, B // NW * L)
    table_fat = _build_fat(table.T)
    out_fat = _make_gather(B, L)(x_flat, table_fat)
    return out_fat[:, :, :EMB]
